# baseline (speedup 1.0000x reference)
import functools

import jax
import jax.numpy as jnp
from jax import lax
from jax.experimental import pallas as pl
from jax.experimental.pallas import tpu as pltpu
from jax.experimental.pallas import tpu_sc as plsc

E = 64
D = 1024
F = 2048
K = 2
T = 2048
A = T * K
TM = 128
NT = A // TM + E
NTP = 128
NPAD = NT * TM

NC = 2
NS = 16
NW = NC * NS
LANES = 16


def _route(expert_ids):
    flat_ids = expert_ids.reshape(-1).astype(jnp.int32)
    sort_idx = jnp.argsort(flat_ids).astype(jnp.int32)
    sorted_ids = jnp.sort(flat_ids)

    counts = jnp.zeros((E,), jnp.int32).at[flat_ids].add(1)
    offsets = jnp.concatenate([jnp.zeros((1,), jnp.int32),
                               jnp.cumsum(counts)[:-1].astype(jnp.int32)])
    ntiles = (counts + TM - 1) // TM
    tile_cum = jnp.concatenate([jnp.zeros((1,), jnp.int32),
                                jnp.cumsum(ntiles)[:-1].astype(jnp.int32)])
    total_tiles = tile_cum[-1] + ntiles[-1]

    t_arr = jnp.arange(NT, dtype=jnp.int32)
    incl = (tile_cum + ntiles).astype(jnp.int32)
    te = jnp.searchsorted(incl, t_arr, side="right").astype(jnp.int32)
    te = jnp.minimum(te, E - 1)
    j = t_arr - tile_cum[te]
    src_start = (offsets[te] + j * TM).astype(jnp.int32)
    tv = jnp.where(t_arr < total_tiles,
                   jnp.clip(counts[te] - j * TM, 0, TM), 0).astype(jnp.int32)

    delta = tile_cum * TM - offsets
    pad_pos_sorted = jnp.arange(A, dtype=jnp.int32) + delta[sorted_ids]
    pos_flat = jnp.zeros((A,), jnp.int32).at[sort_idx].set(pad_pos_sorted)
    posT = pos_flat.reshape(T, K).T
    pad = jnp.zeros((NTP - NT,), jnp.int32)
    src_start_p = jnp.concatenate([src_start, pad])
    tv_p = jnp.concatenate([tv, pad])
    return sort_idx, te, src_start_p, tv_p, posT


_G_PW = NPAD // NW
_G_CH = 48
_G_NC = _G_PW // _G_CH


def _sc_gather(hidden_states, sort_idx, src_start, tv):
    mesh = plsc.VectorSubcoreMesh(core_axis_name="c", subcore_axis_name="s")

    @functools.partial(
        pl.kernel, mesh=mesh,
        out_type=jax.ShapeDtypeStruct((NPAD, 8, 128), jnp.float32),
        compiler_params=pltpu.CompilerParams(needs_layout_passes=False),
        scratch_types=[
            pltpu.VMEM((A,), jnp.int32),
            pltpu.VMEM((NTP,), jnp.int32),
            pltpu.VMEM((NTP,), jnp.int32),
            [pltpu.VMEM((_G_CH,), jnp.int32)] * 2,
            [pltpu.VMEM((_G_CH, 8, 128), jnp.float32)] * 2,
            [pltpu.SemaphoreType.DMA] * 2,
            [pltpu.SemaphoreType.DMA] * 2,
        ],
    )
    def gather_k(hs_hbm, si_hbm, ss_hbm, tv_hbm, out_hbm,
                 si_v, ss_v, tv_v, idx_v, rows_v, gsem, wsem):
        wid = lax.axis_index("s") * NC + lax.axis_index("c")
        base = wid * _G_PW
        pltpu.sync_copy(si_hbm, si_v)
        pltpu.sync_copy(ss_hbm, ss_v)
        pltpu.sync_copy(tv_hbm, tv_v)
        lanes = jnp.arange(LANES, dtype=jnp.int32)

        def compute_idx(c):
            off = base + c * _G_CH
            for k in range(_G_CH // LANES):
                p = off + k * LANES + lanes
                tp = lax.shift_right_logical(p, 7)
                ip = jnp.bitwise_and(p, TM - 1)
                src = plsc.load_gather(ss_v, [tp])
                tvv = plsc.load_gather(tv_v, [tp])
                r = jnp.minimum(src + ip, A - 1)
                f = plsc.load_gather(si_v, [r])
                tok = lax.shift_right_logical(f, 1)
                idx_v[c % 2][pl.ds(k * LANES, LANES)] = (
                    jnp.where(ip < tvv, tok, 0))

        nsub = 6
        sub = _G_CH // nsub

        def start_gather(par):
            return [
                pltpu.async_copy(
                    hs_hbm.at[idx_v[par].at[pl.ds(s * sub, sub)]],
                    rows_v[par].at[pl.ds(s * sub, sub)], gsem[par])
                for s in range(nsub)
            ]

        gcp = [None, None]
        wcp = [None, None]
        compute_idx(0)
        gcp[0] = start_gather(0)
        for c in range(_G_NC):
            par = c % 2
            if c + 1 < _G_NC:
                compute_idx(c + 1)
            for cp in gcp[par]:
                cp.wait()
            wcp[par] = pltpu.async_copy(
                rows_v[par], out_hbm.at[pl.ds(base + c * _G_CH, _G_CH)],
                wsem[par])
            if c + 1 < _G_NC:
                npar = (c + 1) % 2
                if wcp[npar] is not None:
                    wcp[npar].wait()
                gcp[npar] = start_gather(npar)
        wcp[0].wait()
        wcp[1].wait()

    hs3 = hidden_states.reshape(T, 8, 128)
    return gather_k(hs3, sort_idx, src_start, tv).reshape(NPAD, D)


def _mlp_body(te_ref, tv_ref, xs_ref, w1_ref, w2_ref, out_ref):
    @pl.when(tv_ref[pl.program_id(0)] > 0)
    def _():
        x = xs_ref[...]
        h = jax.lax.dot_general(
            x, w1_ref[0], (((1,), (0,)), ((), ())),
            preferred_element_type=jnp.float32)
        h = jnp.maximum(h, 0.0)
        out_ref[...] = jax.lax.dot_general(
            h, w2_ref[0], (((1,), (0,)), ((), ())),
            preferred_element_type=jnp.float32)


def _tc_mlp(te, tv, xs, W1, W2):
    grid_spec = pltpu.PrefetchScalarGridSpec(
        num_scalar_prefetch=2,
        grid=(NT,),
        in_specs=[
            pl.BlockSpec((TM, D), lambda t, te, tv: (t, 0)),
            pl.BlockSpec((1, D, F), lambda t, te, tv: (te[t], 0, 0)),
            pl.BlockSpec((1, F, D), lambda t, te, tv: (te[t], 0, 0)),
        ],
        out_specs=pl.BlockSpec((TM, D), lambda t, te, tv: (t, 0)),
    )
    return pl.pallas_call(
        _mlp_body,
        grid_spec=grid_spec,
        out_shape=jax.ShapeDtypeStruct((NPAD, D), jnp.float32),
    )(te, tv, xs, W1, W2)


_C_PW = T // NW
_C_CH = 32


def _sc_combine(ys, posT, wT):
    mesh = plsc.VectorSubcoreMesh(core_axis_name="c", subcore_axis_name="s")

    @functools.partial(
        pl.kernel, mesh=mesh,
        out_type=jax.ShapeDtypeStruct((T, 8, 128), jnp.float32),
        compiler_params=pltpu.CompilerParams(needs_layout_passes=False),
        scratch_types=[
            pltpu.VMEM((_C_CH,), jnp.int32),
            pltpu.VMEM((_C_CH,), jnp.int32),
            pltpu.VMEM((_C_CH, 8, 128), jnp.float32),
            pltpu.VMEM((_C_CH, 8, 128), jnp.float32),
            pltpu.VMEM((_C_PW,), jnp.float32),
            pltpu.VMEM((_C_PW,), jnp.float32),
            pltpu.SemaphoreType.DMA,
            pltpu.SemaphoreType.DMA,
        ],
    )
    def combine_k(ys_hbm, pos_hbm, w_hbm, out_hbm,
                  i0_v, i1_v, b0, b1, w0_v, w1_v, s0, s1):
        wid = lax.axis_index("s") * NC + lax.axis_index("c")
        base = wid * _C_PW
        pltpu.sync_copy(w_hbm.at[0, pl.ds(base, _C_PW)], w0_v)
        pltpu.sync_copy(w_hbm.at[1, pl.ds(base, _C_PW)], w1_v)
        for c in range(_C_PW // _C_CH):
            off = base + c * _C_CH
            pltpu.sync_copy(pos_hbm.at[0, pl.ds(off, _C_CH)], i0_v)
            pltpu.sync_copy(pos_hbm.at[1, pl.ds(off, _C_CH)], i1_v)
            cp0 = pltpu.async_copy(ys_hbm.at[i0_v], b0, s0)
            cp1 = pltpu.async_copy(ys_hbm.at[i1_v], b1, s1)
            cp0.wait()
            cp1.wait()

            def row_comb(i, carry):
                gi = jnp.full((LANES,), c * _C_CH + i, jnp.int32)
                w0 = plsc.load_gather(w0_v, [gi])
                w1 = plsc.load_gather(w1_v, [gi])

                def sub_comb(jv, carry2):
                    def vec_comb(kv, carry3):
                        sl = pl.ds(kv * LANES, LANES)
                        b0[i, jv, sl] = b0[i, jv, sl] * w0 + b1[i, jv, sl] * w1
                        return carry3
                    return lax.fori_loop(0, 128 // LANES, vec_comb, carry2)
                return lax.fori_loop(0, 8, sub_comb, carry)

            lax.fori_loop(0, _C_CH, row_comb, 0)
            pltpu.sync_copy(b0, out_hbm.at[pl.ds(off, _C_CH)])

    ys3 = ys.reshape(NPAD, 8, 128)
    return combine_k(ys3, posT, wT).reshape(T, D)


def kernel(hidden_states, expert_weights, expert_ids, W1, W2):
    sort_idx, te, src_start, tv, posT = _route(expert_ids)
    wT = expert_weights.T
    xs = _sc_gather(hidden_states, sort_idx, src_start, tv)
    ys = _tc_mlp(te, tv, xs, W1, W2)
    return _sc_combine(ys, posT, wT)


# speedup vs baseline: 1.3316x; 1.3316x over previous
import functools

import jax
import jax.numpy as jnp
from jax import lax
from jax.experimental import pallas as pl
from jax.experimental.pallas import tpu as pltpu
from jax.experimental.pallas import tpu_sc as plsc

E = 64
D = 1024
F = 2048
K = 2
T = 2048
A = T * K
TM = 128
NT = A // TM + E
NTP = 128
NPAD = NT * TM

NC = 2
NS = 16
NW = NC * NS
LANES = 16


def _route(expert_ids):
    flat_ids = expert_ids.reshape(-1).astype(jnp.int32)
    sort_idx = jnp.argsort(flat_ids).astype(jnp.int32)
    sorted_ids = jnp.sort(flat_ids)

    counts = jnp.zeros((E,), jnp.int32).at[flat_ids].add(1)
    offsets = jnp.concatenate([jnp.zeros((1,), jnp.int32),
                               jnp.cumsum(counts)[:-1].astype(jnp.int32)])
    ntiles = (counts + TM - 1) // TM
    tile_cum = jnp.concatenate([jnp.zeros((1,), jnp.int32),
                                jnp.cumsum(ntiles)[:-1].astype(jnp.int32)])
    total_tiles = tile_cum[-1] + ntiles[-1]

    t_arr = jnp.arange(NT, dtype=jnp.int32)
    incl = (tile_cum + ntiles).astype(jnp.int32)
    te = jnp.searchsorted(incl, t_arr, side="right").astype(jnp.int32)
    te = jnp.minimum(te, E - 1)
    j = t_arr - tile_cum[te]
    src_start = (offsets[te] + j * TM).astype(jnp.int32)
    tv = jnp.where(t_arr < total_tiles,
                   jnp.clip(counts[te] - j * TM, 0, TM), 0).astype(jnp.int32)

    delta = tile_cum * TM - offsets
    pad_pos_sorted = jnp.arange(A, dtype=jnp.int32) + delta[sorted_ids]
    inv = jnp.argsort(sort_idx).astype(jnp.int32)
    pos_flat = pad_pos_sorted[inv]
    posT = pos_flat.reshape(T, K).T
    pad = jnp.zeros((NTP - NT,), jnp.int32)
    src_start_p = jnp.concatenate([src_start, pad])
    tv_p = jnp.concatenate([tv, pad])
    nrows = jnp.full((16,), total_tiles * TM, jnp.int32)
    return sort_idx, te, src_start_p, tv_p, posT, nrows


_G_PW = NPAD // NW
_G_CH = 48
_G_NC = _G_PW // _G_CH


def _sc_gather(hidden_states, sort_idx, src_start, tv, nrows_arr):
    mesh = plsc.VectorSubcoreMesh(core_axis_name="c", subcore_axis_name="s")

    @functools.partial(
        pl.kernel, mesh=mesh,
        out_type=jax.ShapeDtypeStruct((NPAD, D), jnp.float32),
        compiler_params=pltpu.CompilerParams(needs_layout_passes=False),
        scratch_types=[
            pltpu.VMEM((A,), jnp.int32),
            pltpu.VMEM((NTP,), jnp.int32),
            pltpu.VMEM((NTP,), jnp.int32),
            pltpu.VMEM((LANES,), jnp.int32),
            [pltpu.VMEM((_G_CH,), jnp.int32)] * 2,
            [pltpu.VMEM((_G_CH, D), jnp.float32)] * 2,
            [pltpu.SemaphoreType.DMA] * 2,
            [pltpu.SemaphoreType.DMA] * 2,
        ],
    )
    def gather_k(hs_hbm, si_hbm, ss_hbm, tv_hbm, nr_hbm, out_hbm,
                 si_v, ss_v, tv_v, nr_v, idx_v, rows_v, gsem, wsem):
        wid = lax.axis_index("s") * NC + lax.axis_index("c")
        base = wid * _G_PW
        pltpu.sync_copy(si_hbm, si_v)
        pltpu.sync_copy(ss_hbm, ss_v)
        pltpu.sync_copy(tv_hbm, tv_v)
        pltpu.sync_copy(nr_hbm, nr_v)
        nrows = nr_v[...][0]
        lanes = jnp.arange(LANES, dtype=jnp.int32)

        def compute_idx(c):
            off = base + c * _G_CH
            for k in range(_G_CH // LANES):
                p = off + k * LANES + lanes
                tp = lax.shift_right_logical(p, 7)
                ip = jnp.bitwise_and(p, TM - 1)
                src = plsc.load_gather(ss_v, [tp])
                tvv = plsc.load_gather(tv_v, [tp])
                r = jnp.minimum(src + ip, A - 1)
                f = plsc.load_gather(si_v, [r])
                tok = lax.shift_right_logical(f, 1)
                idx_v[c % 2][pl.ds(k * LANES, LANES)] = (
                    jnp.where(ip < tvv, tok, 0))

        nsub = 6
        sub = _G_CH // nsub

        def start_gather(par):
            return [
                pltpu.async_copy(
                    hs_hbm.at[idx_v[par].at[pl.ds(s * sub, sub)]],
                    rows_v[par].at[pl.ds(s * sub, sub)], gsem[par])
                for s in range(nsub)
            ]

        def ran(c):
            return base + c * _G_CH < nrows

        wcp = [None, None]
        for c in range(_G_NC):
            par = c % 2
            if wcp[par] is not None:
                wcp[par].wait()

            @pl.when(ran(c))
            def _(c=c, par=par):
                compute_idx(c)
                for cp in start_gather(par):
                    cp.wait()

            wcp[par] = pltpu.async_copy(
                rows_v[par],
                out_hbm.at[pl.ds(base + c * _G_CH, _G_CH)], wsem[par])
        wcp[0].wait()
        wcp[1].wait()

    return gather_k(hidden_states, sort_idx, src_start, tv, nrows_arr)


def _mlp_body(te_ref, tv_ref, xs_ref, w1_ref, w2_ref, out_ref):
    @pl.when(tv_ref[pl.program_id(0)] > 0)
    def _():
        x = xs_ref[...]
        h = jax.lax.dot_general(
            x, w1_ref[0], (((1,), (0,)), ((), ())),
            preferred_element_type=jnp.float32)
        h = jnp.maximum(h, 0.0)
        out_ref[...] = jax.lax.dot_general(
            h, w2_ref[0], (((1,), (0,)), ((), ())),
            preferred_element_type=jnp.float32)


def _tc_mlp(te, tv, xs, W1, W2):
    grid_spec = pltpu.PrefetchScalarGridSpec(
        num_scalar_prefetch=2,
        grid=(NT,),
        in_specs=[
            pl.BlockSpec((TM, D), lambda t, te, tv: (t, 0)),
            pl.BlockSpec((1, D, F), lambda t, te, tv: (te[t], 0, 0)),
            pl.BlockSpec((1, F, D), lambda t, te, tv: (te[t], 0, 0)),
        ],
        out_specs=pl.BlockSpec((TM, D), lambda t, te, tv: (t, 0)),
    )
    return pl.pallas_call(
        _mlp_body,
        grid_spec=grid_spec,
        out_shape=jax.ShapeDtypeStruct((NPAD, D), jnp.float32),
    )(te, tv, xs, W1, W2)


_C_PW = T // NW
_C_CH = 32


def _sc_combine(ys, posT, wT):
    mesh = plsc.VectorSubcoreMesh(core_axis_name="c", subcore_axis_name="s")

    @functools.partial(
        pl.kernel, mesh=mesh,
        out_type=jax.ShapeDtypeStruct((T, D), jnp.float32),
        compiler_params=pltpu.CompilerParams(needs_layout_passes=False),
        scratch_types=[
            pltpu.VMEM((_C_CH,), jnp.int32),
            pltpu.VMEM((_C_CH,), jnp.int32),
            pltpu.VMEM((_C_CH, D), jnp.float32),
            pltpu.VMEM((_C_CH, D), jnp.float32),
            pltpu.VMEM((_C_PW,), jnp.float32),
            pltpu.VMEM((_C_PW,), jnp.float32),
            pltpu.SemaphoreType.DMA,
            pltpu.SemaphoreType.DMA,
        ],
    )
    def combine_k(ys_hbm, pos_hbm, w_hbm, out_hbm,
                  i0_v, i1_v, b0, b1, w0_v, w1_v, s0, s1):
        wid = lax.axis_index("s") * NC + lax.axis_index("c")
        base = wid * _C_PW
        pltpu.sync_copy(w_hbm.at[0, pl.ds(base, _C_PW)], w0_v)
        pltpu.sync_copy(w_hbm.at[1, pl.ds(base, _C_PW)], w1_v)
        for c in range(_C_PW // _C_CH):
            off = base + c * _C_CH
            pltpu.sync_copy(pos_hbm.at[0, pl.ds(off, _C_CH)], i0_v)
            pltpu.sync_copy(pos_hbm.at[1, pl.ds(off, _C_CH)], i1_v)
            cp0 = pltpu.async_copy(ys_hbm.at[i0_v], b0, s0)
            cp1 = pltpu.async_copy(ys_hbm.at[i1_v], b1, s1)
            cp0.wait()
            cp1.wait()

            def row_comb(i, carry):
                gi = jnp.full((LANES,), c * _C_CH + i, jnp.int32)
                w0 = plsc.load_gather(w0_v, [gi])
                w1 = plsc.load_gather(w1_v, [gi])

                def vec_comb(jv, carry2):
                    sl = pl.ds(jv * LANES, LANES)
                    b0[i, sl] = b0[i, sl] * w0 + b1[i, sl] * w1
                    return carry2
                return lax.fori_loop(0, D // LANES, vec_comb, carry)

            lax.fori_loop(0, _C_CH, row_comb, 0)
            pltpu.sync_copy(b0, out_hbm.at[pl.ds(off, _C_CH)])

    return combine_k(ys, posT, wT)


def kernel(hidden_states, expert_weights, expert_ids, W1, W2):
    sort_idx, te, src_start, tv, posT, nrows = _route(expert_ids)
    wT = expert_weights.T
    xs = _sc_gather(hidden_states, sort_idx, src_start, tv, nrows)
    ys = _tc_mlp(te, tv, xs, W1, W2)
    return _sc_combine(ys, posT, wT)


# speedup vs baseline: 1.3323x; 1.0005x over previous
import functools

import jax
import jax.numpy as jnp
from jax import lax
from jax.experimental import pallas as pl
from jax.experimental.pallas import tpu as pltpu
from jax.experimental.pallas import tpu_sc as plsc

E = 64
D = 1024
F = 2048
K = 2
T = 2048
A = T * K
TM = 128
NT = A // TM + E
NTP = 128
NPAD = NT * TM

NC = 2
NS = 16
NW = NC * NS
LANES = 16


def _route(expert_ids):
    flat_ids = expert_ids.reshape(-1).astype(jnp.int32)
    sort_idx = jnp.argsort(flat_ids).astype(jnp.int32)
    sorted_ids = jnp.sort(flat_ids)

    offsets = jnp.searchsorted(
        sorted_ids, jnp.arange(E, dtype=jnp.int32), side="left"
    ).astype(jnp.int32)
    counts = jnp.diff(
        jnp.concatenate([offsets, jnp.full((1,), A, jnp.int32)]))
    ntiles = (counts + TM - 1) // TM
    tile_cum = jnp.concatenate([jnp.zeros((1,), jnp.int32),
                                jnp.cumsum(ntiles)[:-1].astype(jnp.int32)])
    total_tiles = tile_cum[-1] + ntiles[-1]

    t_arr = jnp.arange(NT, dtype=jnp.int32)
    incl = (tile_cum + ntiles).astype(jnp.int32)
    te = jnp.searchsorted(incl, t_arr, side="right").astype(jnp.int32)
    te = jnp.minimum(te, E - 1)
    j = t_arr - tile_cum[te]
    src_start = (offsets[te] + j * TM).astype(jnp.int32)
    tv = jnp.where(t_arr < total_tiles,
                   jnp.clip(counts[te] - j * TM, 0, TM), 0).astype(jnp.int32)

    delta = tile_cum * TM - offsets
    pad_pos_sorted = jnp.arange(A, dtype=jnp.int32) + delta[sorted_ids]
    inv = jnp.argsort(sort_idx).astype(jnp.int32)
    pos_flat = pad_pos_sorted[inv]
    posT = pos_flat.reshape(T, K).T
    pad = jnp.zeros((NTP - NT,), jnp.int32)
    src_start_p = jnp.concatenate([src_start, pad])
    tv_p = jnp.concatenate([tv, pad])
    nrows = jnp.full((16,), total_tiles * TM, jnp.int32)
    return sort_idx, te, src_start_p, tv_p, posT, nrows


_G_PW = NPAD // NW
_G_CH = 48
_G_NC = _G_PW // _G_CH


def _sc_gather(hidden_states, sort_idx, src_start, tv, nrows_arr):
    mesh = plsc.VectorSubcoreMesh(core_axis_name="c", subcore_axis_name="s")

    @functools.partial(
        pl.kernel, mesh=mesh,
        out_type=jax.ShapeDtypeStruct((NPAD, D), jnp.float32),
        compiler_params=pltpu.CompilerParams(needs_layout_passes=False),
        scratch_types=[
            pltpu.VMEM((A,), jnp.int32),
            pltpu.VMEM((NTP,), jnp.int32),
            pltpu.VMEM((NTP,), jnp.int32),
            pltpu.VMEM((LANES,), jnp.int32),
            [pltpu.VMEM((_G_CH,), jnp.int32)] * 2,
            [pltpu.VMEM((_G_CH, D), jnp.float32)] * 2,
            [pltpu.SemaphoreType.DMA] * 2,
            [pltpu.SemaphoreType.DMA] * 2,
        ],
    )
    def gather_k(hs_hbm, si_hbm, ss_hbm, tv_hbm, nr_hbm, out_hbm,
                 si_v, ss_v, tv_v, nr_v, idx_v, rows_v, gsem, wsem):
        wid = lax.axis_index("s") * NC + lax.axis_index("c")
        base = wid * _G_PW
        pltpu.sync_copy(si_hbm, si_v)
        pltpu.sync_copy(ss_hbm, ss_v)
        pltpu.sync_copy(tv_hbm, tv_v)
        pltpu.sync_copy(nr_hbm, nr_v)
        nrows = nr_v[...][0]
        lanes = jnp.arange(LANES, dtype=jnp.int32)

        def compute_idx(c):
            off = base + c * _G_CH
            for k in range(_G_CH // LANES):
                p = off + k * LANES + lanes
                tp = lax.shift_right_logical(p, 7)
                ip = jnp.bitwise_and(p, TM - 1)
                src = plsc.load_gather(ss_v, [tp])
                tvv = plsc.load_gather(tv_v, [tp])
                r = jnp.minimum(src + ip, A - 1)
                f = plsc.load_gather(si_v, [r])
                tok = lax.shift_right_logical(f, 1)
                idx_v[c % 2][pl.ds(k * LANES, LANES)] = (
                    jnp.where(ip < tvv, tok, 0))

        nsub = 6
        sub = _G_CH // nsub

        def start_gather(par):
            return [
                pltpu.async_copy(
                    hs_hbm.at[idx_v[par].at[pl.ds(s * sub, sub)]],
                    rows_v[par].at[pl.ds(s * sub, sub)], gsem[par])
                for s in range(nsub)
            ]

        def ran(c):
            return base + c * _G_CH < nrows

        wcp = [None, None]
        for c in range(_G_NC):
            par = c % 2
            if wcp[par] is not None:
                wcp[par].wait()

            @pl.when(ran(c))
            def _(c=c, par=par):
                compute_idx(c)
                for cp in start_gather(par):
                    cp.wait()

            wcp[par] = pltpu.async_copy(
                rows_v[par],
                out_hbm.at[pl.ds(base + c * _G_CH, _G_CH)], wsem[par])
        wcp[0].wait()
        wcp[1].wait()

    return gather_k(hidden_states, sort_idx, src_start, tv, nrows_arr)


def _mlp_body(te_ref, tv_ref, xs_ref, w1_ref, w2_ref, out_ref):
    @pl.when(tv_ref[pl.program_id(0)] > 0)
    def _():
        x = xs_ref[...]
        h = jax.lax.dot_general(
            x, w1_ref[0], (((1,), (0,)), ((), ())),
            preferred_element_type=jnp.float32)
        h = jnp.maximum(h, 0.0)
        out_ref[...] = jax.lax.dot_general(
            h, w2_ref[0], (((1,), (0,)), ((), ())),
            preferred_element_type=jnp.float32)


def _tc_mlp(te, tv, xs, W1, W2):
    grid_spec = pltpu.PrefetchScalarGridSpec(
        num_scalar_prefetch=2,
        grid=(NT,),
        in_specs=[
            pl.BlockSpec((TM, D), lambda t, te, tv: (t, 0)),
            pl.BlockSpec((1, D, F), lambda t, te, tv: (te[t], 0, 0)),
            pl.BlockSpec((1, F, D), lambda t, te, tv: (te[t], 0, 0)),
        ],
        out_specs=pl.BlockSpec((TM, D), lambda t, te, tv: (t, 0)),
    )
    return pl.pallas_call(
        _mlp_body,
        grid_spec=grid_spec,
        out_shape=jax.ShapeDtypeStruct((NPAD, D), jnp.float32),
    )(te, tv, xs, W1, W2)


_C_PW = T // NW
_C_CH = 32


def _sc_combine(ys, posT, wT):
    mesh = plsc.VectorSubcoreMesh(core_axis_name="c", subcore_axis_name="s")

    @functools.partial(
        pl.kernel, mesh=mesh,
        out_type=jax.ShapeDtypeStruct((T, D), jnp.float32),
        compiler_params=pltpu.CompilerParams(needs_layout_passes=False),
        scratch_types=[
            pltpu.VMEM((_C_CH,), jnp.int32),
            pltpu.VMEM((_C_CH,), jnp.int32),
            pltpu.VMEM((_C_CH, D), jnp.float32),
            pltpu.VMEM((_C_CH, D), jnp.float32),
            pltpu.VMEM((_C_PW,), jnp.float32),
            pltpu.VMEM((_C_PW,), jnp.float32),
            pltpu.SemaphoreType.DMA,
            pltpu.SemaphoreType.DMA,
        ],
    )
    def combine_k(ys_hbm, pos_hbm, w_hbm, out_hbm,
                  i0_v, i1_v, b0, b1, w0_v, w1_v, s0, s1):
        wid = lax.axis_index("s") * NC + lax.axis_index("c")
        base = wid * _C_PW
        pltpu.sync_copy(w_hbm.at[0, pl.ds(base, _C_PW)], w0_v)
        pltpu.sync_copy(w_hbm.at[1, pl.ds(base, _C_PW)], w1_v)
        for c in range(_C_PW // _C_CH):
            off = base + c * _C_CH
            pltpu.sync_copy(pos_hbm.at[0, pl.ds(off, _C_CH)], i0_v)
            pltpu.sync_copy(pos_hbm.at[1, pl.ds(off, _C_CH)], i1_v)
            cp0 = pltpu.async_copy(ys_hbm.at[i0_v], b0, s0)
            cp1 = pltpu.async_copy(ys_hbm.at[i1_v], b1, s1)
            cp0.wait()
            cp1.wait()

            def row_comb(i, carry):
                gi = jnp.full((LANES,), c * _C_CH + i, jnp.int32)
                w0 = plsc.load_gather(w0_v, [gi])
                w1 = plsc.load_gather(w1_v, [gi])

                def vec_comb(jv, carry2):
                    sl = pl.ds(jv * LANES, LANES)
                    b0[i, sl] = b0[i, sl] * w0 + b1[i, sl] * w1
                    return carry2
                return lax.fori_loop(0, D // LANES, vec_comb, carry)

            lax.fori_loop(0, _C_CH, row_comb, 0)
            pltpu.sync_copy(b0, out_hbm.at[pl.ds(off, _C_CH)])

    return combine_k(ys, posT, wT)


def kernel(hidden_states, expert_weights, expert_ids, W1, W2):
    sort_idx, te, src_start, tv, posT, nrows = _route(expert_ids)
    wT = expert_weights.T
    xs = _sc_gather(hidden_states, sort_idx, src_start, tv, nrows)
    ys = _tc_mlp(te, tv, xs, W1, W2)
    return _sc_combine(ys, posT, wT)
